# baseline (speedup 1.0000x reference)
import jax
import jax.numpy as jnp
from jax.experimental import pallas as pl

_M = 8192
_N = 16384
_C = 128
_NQ = 128

_VS = (0.05, 0.05, 0.1)


def _nn_interp_kernel(q_ref, xiT_ref, feat_ref, out_ref):
    xiT = xiT_ref[...].astype(jnp.float32)
    kb = xiT[0:1, :]
    kx = xiT[3:4, :] * _VS[0] + (1.5 * _VS[0])
    ky = xiT[2:3, :] * _VS[1] + (1.5 * _VS[1])
    kz = xiT[1:2, :] * _VS[2] + (1.5 * _VS[2])
    kk = kb * kb + kx * kx + ky * ky + kz * kz
    ones_m = jnp.ones((1, _M), jnp.float32)
    kaug = jnp.concatenate(
        [-2.0 * kb, -2.0 * kx, -2.0 * ky, -2.0 * kz, kk, ones_m,
         jnp.zeros((2, _M), jnp.float32)], axis=0)

    q = q_ref[...]
    qq = jnp.sum(q * q, axis=1, keepdims=True)
    qaug = jnp.concatenate(
        [q, jnp.ones((_NQ, 1), jnp.float32), qq,
         jnp.zeros((_NQ, 2), jnp.float32)], axis=1)

    d2 = jnp.dot(qaug, kaug, preferred_element_type=jnp.float32,
                 precision=jax.lax.Precision.HIGHEST)
    d2 = jnp.maximum(d2, 0.0)

    lane = jax.lax.broadcasted_iota(jnp.int32, (_NQ, _M), 1).astype(jnp.float32)
    big = jnp.float32(1e30)
    bigl = jnp.float32(_M)

    def pass_j(d, mask_after):
        m = jnp.min(d, axis=1, keepdims=True)
        i = jnp.min(jnp.where(d == m, lane, bigl), axis=1, keepdims=True)
        eqm = lane == i
        if mask_after:
            d = jnp.where(eqm, big, d)
        return d, m, eqm

    d2, m0, e0 = pass_j(d2, True)
    d2, m1, e1 = pass_j(d2, True)
    _, m2, e2 = pass_j(d2, False)

    r0 = 1.0 / (m0 + 1e-8)
    r1 = 1.0 / (m1 + 1e-8)
    r2 = 1.0 / (m2 + 1e-8)
    inv_norm = 1.0 / (r0 + r1 + r2)
    w0 = r0 * inv_norm
    w1 = r1 * inv_norm
    w2 = r2 * inv_norm

    zero = jnp.zeros((), jnp.float32)
    w = jnp.where(e0, w0, jnp.where(e1, w1, jnp.where(e2, w2, zero)))

    out_ref[...] = jnp.dot(w, feat_ref[...],
                           preferred_element_type=jnp.float32,
                           precision=jax.lax.Precision.HIGHEST)


def kernel(x_features, x_indices, points_mean):
    xiT = x_indices.astype(jnp.int32).T

    grid = (_N // _NQ,)
    out = pl.pallas_call(
        _nn_interp_kernel,
        grid=grid,
        in_specs=[
            pl.BlockSpec((_NQ, 4), lambda i: (i, 0)),
            pl.BlockSpec((4, _M), lambda i: (0, 0)),
            pl.BlockSpec((_M, _C), lambda i: (0, 0)),
        ],
        out_specs=pl.BlockSpec((_NQ, _C), lambda i: (i, 0)),
        out_shape=jax.ShapeDtypeStruct((_N, _C), jnp.float32),
    )(points_mean, xiT, x_features)
    return out


# speedup vs baseline: 1.7019x; 1.7019x over previous
import jax
import jax.numpy as jnp
from jax.experimental import pallas as pl

_M = 8192
_N = 16384
_C = 128
_NQ = 128

_VS = (0.05, 0.05, 0.1)


def _nn_interp_kernel(q_ref, xiT_ref, feat_ref, out_ref):
    xiT = xiT_ref[...].astype(jnp.float32)
    kb = xiT[0:1, :]
    kx = xiT[3:4, :] * _VS[0] + (1.5 * _VS[0])
    ky = xiT[2:3, :] * _VS[1] + (1.5 * _VS[1])
    kz = xiT[1:2, :] * _VS[2] + (1.5 * _VS[2])
    kk = kb * kb + kx * kx + ky * ky + kz * kz

    q = q_ref[...]
    qq = jnp.sum(q * q, axis=1, keepdims=True)
    cross = (q[:, 0:1] * kb + q[:, 1:2] * kx
             + q[:, 2:3] * ky + q[:, 3:4] * kz)
    d2 = jnp.maximum((qq + kk) - 2.0 * cross, 0.0)

    lane = jax.lax.broadcasted_iota(jnp.int32, (_NQ, _M), 1).astype(jnp.float32)
    big = jnp.float32(1e30)
    bigl = jnp.float32(_M)

    def pass_j(d, mask_after):
        m = jnp.min(d, axis=1, keepdims=True)
        i = jnp.min(jnp.where(d == m, lane, bigl), axis=1, keepdims=True)
        eqm = lane == i
        if mask_after:
            d = jnp.where(eqm, big, d)
        return d, m, eqm

    d2, m0, e0 = pass_j(d2, True)
    d2, m1, e1 = pass_j(d2, True)
    _, m2, e2 = pass_j(d2, False)

    r0 = 1.0 / (m0 + 1e-8)
    r1 = 1.0 / (m1 + 1e-8)
    r2 = 1.0 / (m2 + 1e-8)
    inv_norm = 1.0 / (r0 + r1 + r2)
    w0 = r0 * inv_norm
    w1 = r1 * inv_norm
    w2 = r2 * inv_norm

    zero = jnp.zeros((), jnp.float32)
    w = jnp.where(e0, w0, jnp.where(e1, w1, jnp.where(e2, w2, zero)))

    out_ref[...] = jnp.dot(w, feat_ref[...],
                           preferred_element_type=jnp.float32)


def kernel(x_features, x_indices, points_mean):
    xiT = x_indices.astype(jnp.int32).T

    grid = (_N // _NQ,)
    out = pl.pallas_call(
        _nn_interp_kernel,
        grid=grid,
        in_specs=[
            pl.BlockSpec((_NQ, 4), lambda i: (i, 0)),
            pl.BlockSpec((4, _M), lambda i: (0, 0)),
            pl.BlockSpec((_M, _C), lambda i: (0, 0)),
        ],
        out_specs=pl.BlockSpec((_NQ, _C), lambda i: (i, 0)),
        out_shape=jax.ShapeDtypeStruct((_N, _C), jnp.float32),
    )(points_mean, xiT, x_features)
    return out


# speedup vs baseline: 2.1379x; 1.2562x over previous
import jax
import jax.numpy as jnp
from jax.experimental import pallas as pl

_M = 8192
_N = 16384
_C = 128
_NQ = 128

_VS = (0.05, 0.05, 0.1)
_OFF = (0.05, 0.05, 0.1)


def _nn_interp_kernel(q_ref, xiT_ref, feat_ref, out_ref):
    xiT = xiT_ref[...].astype(jnp.float32)
    kb = xiT[0:1, :]
    kx = (xiT[3:4, :] * _VS[0] + _OFF[0]) + 0.5 * _VS[0]
    ky = (xiT[2:3, :] * _VS[1] + _OFF[1]) + 0.5 * _VS[1]
    kz = (xiT[1:2, :] * _VS[2] + _OFF[2]) + 0.5 * _VS[2]
    kk = ((kb * kb + kx * kx) + ky * ky) + kz * kz
    kT = jnp.concatenate([kb, kx, ky, kz], axis=0)

    q = q_ref[...]
    q0 = q[:, 0:1]
    q1 = q[:, 1:2]
    q2 = q[:, 2:3]
    q3 = q[:, 3:4]
    qq = ((q0 * q0 + q1 * q1) + q2 * q2) + q3 * q3

    cross = jnp.dot(q, kT, preferred_element_type=jnp.float32)
    d2 = jnp.maximum((qq + kk) - 2.0 * cross, 0.0)

    lane = jax.lax.broadcasted_iota(jnp.int32, (_NQ, _M), 1).astype(jnp.float32)
    big = jnp.float32(1e30)
    bigl = jnp.float32(_M)

    def pass_j(d, mask_after):
        m = jnp.min(d, axis=1, keepdims=True)
        i = jnp.min(jnp.where(d == m, lane, bigl), axis=1, keepdims=True)
        eqm = lane == i
        if mask_after:
            d = jnp.where(eqm, big, d)
        return d, m, eqm

    d2, m0, e0 = pass_j(d2, True)
    d2, m1, e1 = pass_j(d2, True)
    _, m2, e2 = pass_j(d2, False)

    r0 = 1.0 / (m0 + 1e-8)
    r1 = 1.0 / (m1 + 1e-8)
    r2 = 1.0 / (m2 + 1e-8)
    norm = (r0 + r1) + r2
    w0 = r0 / norm
    w1 = r1 / norm
    w2 = r2 / norm

    zero = jnp.zeros((), jnp.float32)
    w = jnp.where(e0, w0, jnp.where(e1, w1, jnp.where(e2, w2, zero)))

    out_ref[...] = jnp.dot(w, feat_ref[...],
                           preferred_element_type=jnp.float32)


def kernel(x_features, x_indices, points_mean):
    xiT = x_indices.astype(jnp.int32).T

    grid = (_N // _NQ,)
    out = pl.pallas_call(
        _nn_interp_kernel,
        grid=grid,
        in_specs=[
            pl.BlockSpec((_NQ, 4), lambda i: (i, 0)),
            pl.BlockSpec((4, _M), lambda i: (0, 0)),
            pl.BlockSpec((_M, _C), lambda i: (0, 0)),
        ],
        out_specs=pl.BlockSpec((_NQ, _C), lambda i: (i, 0)),
        out_shape=jax.ShapeDtypeStruct((_N, _C), jnp.float32),
    )(points_mean, xiT, x_features)
    return out


# speedup vs baseline: 2.5213x; 1.1793x over previous
import jax
import jax.numpy as jnp
from jax.experimental import pallas as pl

_M = 8192
_N = 16384
_C = 128
_NQ = 128

_VS = (0.05, 0.05, 0.1)
_OFF = (0.05, 0.05, 0.1)


def _nn_interp_kernel(q_ref, xiT_ref, feat_ref, out_ref):
    xiT = xiT_ref[...].astype(jnp.float32)
    kb = xiT[0:1, :]
    kx = (xiT[3:4, :] * _VS[0] + _OFF[0]) + 0.5 * _VS[0]
    ky = (xiT[2:3, :] * _VS[1] + _OFF[1]) + 0.5 * _VS[1]
    kz = (xiT[1:2, :] * _VS[2] + _OFF[2]) + 0.5 * _VS[2]
    kk = ((kb * kb + kx * kx) + ky * ky) + kz * kz
    kT = jnp.concatenate([kb, kx, ky, kz], axis=0)

    q = q_ref[...]
    q0 = q[:, 0:1]
    q1 = q[:, 1:2]
    q2 = q[:, 2:3]
    q3 = q[:, 3:4]
    qq = ((q0 * q0 + q1 * q1) + q2 * q2) + q3 * q3

    cross = jnp.dot(q, kT, preferred_element_type=jnp.float32)
    d2 = jnp.maximum((qq + kk) - 2.0 * cross, 0.0)

    big = jnp.float32(1e30)

    m0 = jnp.min(d2, axis=1, keepdims=True)
    e0 = d2 == m0
    d2 = jnp.where(e0, big, d2)
    m1 = jnp.min(d2, axis=1, keepdims=True)
    e1 = d2 == m1
    d2 = jnp.where(e1, big, d2)
    m2 = jnp.min(d2, axis=1, keepdims=True)
    e2 = d2 == m2

    r0 = 1.0 / (m0 + 1e-8)
    r1 = 1.0 / (m1 + 1e-8)
    r2 = 1.0 / (m2 + 1e-8)

    zero = jnp.zeros((), jnp.float32)
    wr = jnp.where(e0, r0, jnp.where(e1, r1, jnp.where(e2, r2, zero)))
    norm = jnp.sum(wr, axis=1, keepdims=True)
    w = wr * (1.0 / norm)

    out_ref[...] = jnp.dot(w, feat_ref[...],
                           preferred_element_type=jnp.float32)


def kernel(x_features, x_indices, points_mean):
    xiT = x_indices.astype(jnp.int32).T

    grid = (_N // _NQ,)
    out = pl.pallas_call(
        _nn_interp_kernel,
        grid=grid,
        in_specs=[
            pl.BlockSpec((_NQ, 4), lambda i: (i, 0)),
            pl.BlockSpec((4, _M), lambda i: (0, 0)),
            pl.BlockSpec((_M, _C), lambda i: (0, 0)),
        ],
        out_specs=pl.BlockSpec((_NQ, _C), lambda i: (i, 0)),
        out_shape=jax.ShapeDtypeStruct((_N, _C), jnp.float32),
    )(points_mean, xiT, x_features)
    return out
